# Initial kernel scaffold; baseline (speedup 1.0000x reference)
import jax
import jax.numpy as jnp
from jax.experimental import pallas as pl


def kernel(x, table):
    raise NotImplementedError("write your pallas kernel here")


# speedup vs baseline: 1.8651x; 1.8651x over previous
import functools

import jax
import jax.numpy as jnp
from jax.experimental import pallas as pl

SEQ = 4096
BATCH = 4
D_MODEL = 1024

_BS = 256


def _pe_kernel(x_ref, t_ref, o_ref):
    o_ref[...] = x_ref[...] + t_ref[...][:, None, :]


@functools.partial(jax.jit, static_argnames=())
def kernel(x, table):
    s, b, d = x.shape
    grid = (s // _BS,)
    return pl.pallas_call(
        _pe_kernel,
        grid=grid,
        in_specs=[
            pl.BlockSpec((_BS, b, d), lambda i: (i, 0, 0)),
            pl.BlockSpec((_BS, d), lambda i: (i, 0)),
        ],
        out_specs=pl.BlockSpec((_BS, b, d), lambda i: (i, 0, 0)),
        out_shape=jax.ShapeDtypeStruct((s, b, d), x.dtype),
    )(x, table)


# speedup vs baseline: 1.8962x; 1.0167x over previous
import functools

import jax
import jax.numpy as jnp
from jax.experimental import pallas as pl

SEQ = 4096
BATCH = 4
D_MODEL = 1024

_BS = 512


def _pe_kernel(x_ref, t_ref, o_ref):
    o_ref[...] = x_ref[...] + t_ref[...][:, None, :]


@functools.partial(jax.jit, static_argnames=())
def kernel(x, table):
    s, b, d = x.shape
    grid = (s // _BS,)
    return pl.pallas_call(
        _pe_kernel,
        grid=grid,
        in_specs=[
            pl.BlockSpec((_BS, b, d), lambda i: (i, 0, 0)),
            pl.BlockSpec((_BS, d), lambda i: (i, 0)),
        ],
        out_specs=pl.BlockSpec((_BS, b, d), lambda i: (i, 0, 0)),
        out_shape=jax.ShapeDtypeStruct((s, b, d), x.dtype),
    )(x, table)
